# baseline (speedup 1.0000x reference)
import functools

import jax
import jax.numpy as jnp
from jax import lax
from jax.experimental import pallas as pl
from jax.experimental.pallas import tpu as pltpu
from jax.experimental.pallas import tpu_sc as plsc

NC, NS = 2, 16
NW = NC * NS
LANES = 16

BATCH = 1024
N_INPUTS = 4096
N_GATES = 16384
W = BATCH // 32
PACK_CB = 512


def _pack_body(x_ref, out_ref):
    xb = x_ref[...]
    x3 = xb.reshape(32, 32, PACK_CB)
    j = lax.broadcasted_iota(jnp.int32, x3.shape, 1)
    out_ref[...] = jnp.sum(x3 << j, axis=1)


_pack = pl.pallas_call(
    _pack_body,
    grid=(N_INPUTS // PACK_CB,),
    in_specs=[pl.BlockSpec((BATCH, PACK_CB), lambda i: (0, i))],
    out_specs=pl.BlockSpec((W, PACK_CB), lambda i: (0, i)),
    out_shape=jax.ShapeDtypeStruct((W, N_INPUTS), jnp.int32),
)


UNPACK_GB = 2048


def _unpack_body(p_ref, out_ref):
    pk = p_ref[...]
    b3 = jnp.broadcast_to(pk[:, None, :], (W, 32, UNPACK_GB))
    j = lax.broadcasted_iota(jnp.int32, b3.shape, 1)
    out_ref[...] = (((b3 >> j) & 1) != 0).reshape(BATCH, UNPACK_GB)


_unpack = pl.pallas_call(
    _unpack_body,
    grid=(N_GATES // UNPACK_GB,),
    in_specs=[pl.BlockSpec((W, UNPACK_GB), lambda i: (0, i))],
    out_specs=pl.BlockSpec((BATCH, UNPACK_GB), lambda i: (0, i)),
    out_shape=jax.ShapeDtypeStruct((BATCH, N_GATES), jnp.bool_),
)


@functools.cache
def _make_sc_gate():
    mesh = plsc.VectorSubcoreMesh(
        core_axis_name="c", subcore_axis_name="s", num_cores=NC, num_subcores=NS)
    return functools.partial(
        pl.kernel,
        out_type=jax.ShapeDtypeStruct((W, N_GATES), jnp.int32),
        mesh=mesh,
        compiler_params=pltpu.CompilerParams(
            use_tc_tiling_on_sc=False, needs_layout_passes=False),
        scratch_types=[
            pltpu.VMEM((N_INPUTS,), jnp.int32),
            pltpu.VMEM((N_GATES,), jnp.int32),
            pltpu.VMEM((N_GATES,), jnp.int32),
            pltpu.SemaphoreType.DMA,
        ],
    )(_sc_gate_body)


def _sc_gate_body(xp_hbm, ctl_hbm, out_hbm, xrow_v, ctl_v, out_v, sem):
    w = lax.axis_index("s") * NC + lax.axis_index("c")
    d0 = pltpu.async_copy(xp_hbm.at[w], xrow_v, sem)
    d1 = pltpu.async_copy(ctl_hbm.at[:], ctl_v, sem)
    d0.wait()
    d1.wait()

    @plsc.parallel_loop(0, N_GATES // LANES, 1, unroll=8)
    def _loop(q):
        s = q * LANES
        ctl = ctl_v[pl.ds(s, LANES)]
        i0 = ctl & 0xFFF
        i1 = (ctl >> 12) & 0xFFF
        tt = ctl >> 24
        av = plsc.load_gather(xrow_v, [i0])
        bv = plsc.load_gather(xrow_v, [i1])
        t0 = -(tt & 1)
        t1 = -((tt >> 1) & 1)
        t2 = -((tt >> 2) & 1)
        t3 = -((tt >> 3) & 1)
        out_v[pl.ds(s, LANES)] = t0 ^ (av & t2) ^ (bv & t1) ^ (av & bv & t3)
    pltpu.sync_copy(out_v, out_hbm.at[w])


def kernel(x, gates, choices):
    xp = _pack(x)
    g = gates.astype(jnp.int32)
    t0 = g[:, 0]
    t1 = g[:, 0] ^ g[:, 1]
    t2 = g[:, 0] ^ g[:, 2]
    t3 = g[:, 0] ^ g[:, 1] ^ g[:, 2] ^ g[:, 3]
    tt = t0 | (t1 << 1) | (t2 << 2) | (t3 << 3)
    ctl = choices[:, 0] | (choices[:, 1] << 12) | (tt << 24)
    pkw = _make_sc_gate()(xp, ctl)
    return _unpack(pkw)


# speedup vs baseline: 1.3496x; 1.3496x over previous
import functools

import jax
import jax.numpy as jnp
from jax import lax
from jax.experimental import pallas as pl
from jax.experimental.pallas import tpu as pltpu
from jax.experimental.pallas import tpu_sc as plsc

NC, NS = 2, 16
NW = NC * NS
LANES = 16

BATCH = 1024
N_INPUTS = 4096
N_GATES = 16384
W = BATCH // 32
PACK_CB = 512


def _pack_body(x_ref, out_ref):
    xb = x_ref[...]
    x3 = xb.reshape(32, 32, PACK_CB)
    j = lax.broadcasted_iota(jnp.int32, x3.shape, 1)
    out_ref[...] = jnp.sum(x3 << j, axis=1)


_pack = pl.pallas_call(
    _pack_body,
    grid=(N_INPUTS // PACK_CB,),
    in_specs=[pl.BlockSpec((BATCH, PACK_CB), lambda i: (0, i))],
    out_specs=pl.BlockSpec((W, PACK_CB), lambda i: (0, i)),
    out_shape=jax.ShapeDtypeStruct((W, N_INPUTS), jnp.int32),
)


UNPACK_GB = 2048


def _unpack_body(p_ref, out_ref):
    pk = p_ref[...]
    b3 = jnp.broadcast_to(pk[:, None, :], (W, 32, UNPACK_GB))
    j = lax.broadcasted_iota(jnp.int32, b3.shape, 1)
    out_ref[...] = ((b3 >> j) & 1).astype(jnp.int8).reshape(BATCH, UNPACK_GB)


_unpack = pl.pallas_call(
    _unpack_body,
    grid=(N_GATES // UNPACK_GB,),
    in_specs=[pl.BlockSpec((W, UNPACK_GB), lambda i: (0, i))],
    out_specs=pl.BlockSpec((BATCH, UNPACK_GB), lambda i: (0, i)),
    out_shape=jax.ShapeDtypeStruct((BATCH, N_GATES), jnp.int8),
)


@functools.cache
def _make_sc_gate():
    mesh = plsc.VectorSubcoreMesh(
        core_axis_name="c", subcore_axis_name="s", num_cores=NC, num_subcores=NS)
    return functools.partial(
        pl.kernel,
        out_type=jax.ShapeDtypeStruct((W, N_GATES), jnp.int32),
        mesh=mesh,
        compiler_params=pltpu.CompilerParams(
            use_tc_tiling_on_sc=False, needs_layout_passes=False),
        scratch_types=[
            pltpu.VMEM((N_INPUTS,), jnp.int32),
            pltpu.VMEM((N_GATES,), jnp.int32),
            pltpu.VMEM((N_GATES,), jnp.int32),
            pltpu.SemaphoreType.DMA,
        ],
    )(_sc_gate_body)


def _sc_gate_body(xp_hbm, ctl_hbm, out_hbm, xrow_v, ctl_v, out_v, sem):
    w = lax.axis_index("s") * NC + lax.axis_index("c")
    d0 = pltpu.async_copy(xp_hbm.at[w], xrow_v, sem)
    d1 = pltpu.async_copy(ctl_hbm.at[:], ctl_v, sem)
    d0.wait()
    d1.wait()

    @plsc.parallel_loop(0, N_GATES // LANES, 1, unroll=8)
    def _loop(q):
        s = q * LANES
        ctl = ctl_v[pl.ds(s, LANES)]
        i0 = ctl & 0xFFF
        i1 = (ctl >> 12) & 0xFFF
        tt = ctl >> 24
        av = plsc.load_gather(xrow_v, [i0])
        bv = plsc.load_gather(xrow_v, [i1])
        t0 = -(tt & 1)
        t1 = -((tt >> 1) & 1)
        t2 = -((tt >> 2) & 1)
        t3 = -((tt >> 3) & 1)
        out_v[pl.ds(s, LANES)] = t0 ^ (av & t2) ^ (bv & (t1 ^ (av & t3)))
    pltpu.sync_copy(out_v, out_hbm.at[w])


def kernel(x, gates, choices):
    xp = _pack(x)
    g = gates.astype(jnp.int32)
    t0 = g[:, 0]
    t1 = g[:, 0] ^ g[:, 1]
    t2 = g[:, 0] ^ g[:, 2]
    t3 = g[:, 0] ^ g[:, 1] ^ g[:, 2] ^ g[:, 3]
    tt = t0 | (t1 << 1) | (t2 << 2) | (t3 << 3)
    ctl = choices[:, 0] | (choices[:, 1] << 12) | (tt << 24)
    pkw = _make_sc_gate()(xp, ctl)
    out8 = _unpack(pkw)
    return out8.view(jnp.bool_)


# speedup vs baseline: 1.3699x; 1.0150x over previous
import functools

import jax
import jax.numpy as jnp
from jax import lax
from jax.experimental import pallas as pl
from jax.experimental.pallas import tpu as pltpu
from jax.experimental.pallas import tpu_sc as plsc

NC, NS = 2, 16
NW = NC * NS
LANES = 16

BATCH = 1024
N_INPUTS = 4096
N_GATES = 16384
W = BATCH // 32
PACK_CB = 512


def _pack_body(x_ref, out_ref):
    xb = x_ref[...]
    x3 = xb.reshape(32, 32, PACK_CB)
    j = lax.broadcasted_iota(jnp.int32, x3.shape, 1)
    out_ref[...] = jnp.sum(x3 << j, axis=1)


_pack = pl.pallas_call(
    _pack_body,
    grid=(N_INPUTS // PACK_CB,),
    in_specs=[pl.BlockSpec((BATCH, PACK_CB), lambda i: (0, i))],
    out_specs=pl.BlockSpec((W, PACK_CB), lambda i: (0, i)),
    out_shape=jax.ShapeDtypeStruct((W, N_INPUTS), jnp.int32),
)


UNPACK_GB = 4096


def _unpack_body(p_ref, out_ref):
    pk = p_ref[...]
    b3 = jnp.broadcast_to(pk[:, None, :], (W, 32, UNPACK_GB))
    j = lax.broadcasted_iota(jnp.int32, b3.shape, 1)
    out_ref[...] = ((b3 >> j) & 1).astype(jnp.int8).reshape(BATCH, UNPACK_GB)


_unpack = pl.pallas_call(
    _unpack_body,
    grid=(N_GATES // UNPACK_GB,),
    in_specs=[pl.BlockSpec((W, UNPACK_GB), lambda i: (0, i))],
    out_specs=pl.BlockSpec((BATCH, UNPACK_GB), lambda i: (0, i)),
    out_shape=jax.ShapeDtypeStruct((BATCH, N_GATES), jnp.int8),
)


@functools.cache
def _make_sc_gate():
    mesh = plsc.VectorSubcoreMesh(
        core_axis_name="c", subcore_axis_name="s", num_cores=NC, num_subcores=NS)
    return functools.partial(
        pl.kernel,
        out_type=jax.ShapeDtypeStruct((W, N_GATES), jnp.int32),
        mesh=mesh,
        compiler_params=pltpu.CompilerParams(
            use_tc_tiling_on_sc=False, needs_layout_passes=False),
        scratch_types=[
            pltpu.VMEM((N_INPUTS,), jnp.int32),
            pltpu.VMEM((N_GATES,), jnp.int32),
            pltpu.VMEM((N_GATES,), jnp.int32),
            pltpu.SemaphoreType.DMA,
        ],
    )(_sc_gate_body)


def _sc_gate_body(xp_hbm, ctl_hbm, out_hbm, xrow_v, ctl_v, out_v, sem):
    w = lax.axis_index("s") * NC + lax.axis_index("c")
    d0 = pltpu.async_copy(xp_hbm.at[w], xrow_v, sem)
    d1 = pltpu.async_copy(ctl_hbm.at[:], ctl_v, sem)
    d0.wait()
    d1.wait()

    @plsc.parallel_loop(0, N_GATES // LANES, 1, unroll=8)
    def _loop(q):
        s = q * LANES
        ctl = ctl_v[pl.ds(s, LANES)]
        i0 = ctl & 0xFFF
        i1 = (ctl >> 12) & 0xFFF
        tt = ctl >> 24
        av = plsc.load_gather(xrow_v, [i0])
        bv = plsc.load_gather(xrow_v, [i1])
        t0 = -(tt & 1)
        t1 = -((tt >> 1) & 1)
        t2 = -((tt >> 2) & 1)
        t3 = -((tt >> 3) & 1)
        out_v[pl.ds(s, LANES)] = t0 ^ (av & t2) ^ (bv & (t1 ^ (av & t3)))
    pltpu.sync_copy(out_v, out_hbm.at[w])


def kernel(x, gates, choices):
    xp = _pack(x)
    g = gates.astype(jnp.int32)
    t0 = g[:, 0]
    t1 = g[:, 0] ^ g[:, 1]
    t2 = g[:, 0] ^ g[:, 2]
    t3 = g[:, 0] ^ g[:, 1] ^ g[:, 2] ^ g[:, 3]
    tt = t0 | (t1 << 1) | (t2 << 2) | (t3 << 3)
    ctl = choices[:, 0] | (choices[:, 1] << 12) | (tt << 24)
    pkw = _make_sc_gate()(xp, ctl)
    out8 = _unpack(pkw)
    return out8.view(jnp.bool_)
